# baseline (speedup 1.0000x reference)
import jax
import jax.numpy as jnp
from jax.experimental import pallas as pl
from jax.experimental.pallas import tpu as pltpu

_N = 8
_C = 128
_CH = 64


def _route_body(ic_ref, wf_any, keep_ref, cam_emb_ref, w1t_ref, b1_ref,
                w2t_ref, b2_ref, wpt_ref,
                ce_ln_ref, pred_ln_ref, cph_ref, idx_ref, buf, sems):
    ic = ic_ref[0]
    copies = []
    for i in range(8):
        b, h = i // 2, i % 2
        cp = pltpu.make_async_copy(
            wf_any.at[b, ic, pl.ds(h * _CH, _CH)], buf.at[i], sems.at[i])
        cp.start()
        copies.append(cp)

    def _ln(v):
        m = jnp.mean(v, axis=-1, keepdims=True)
        var = jnp.mean((v - m) ** 2, axis=-1, keepdims=True)
        return (v - m) / jnp.sqrt(var + 1e-5)

    ce = cam_emb_ref[...]
    row_sel = (jax.lax.broadcasted_iota(jnp.int32, (_N, 1), 0) == ic)
    ce_row = jnp.sum(jnp.where(row_sel, ce, 0.0), axis=0)[None, :]
    ce_ln = _ln(ce_row)
    col = jax.lax.broadcasted_iota(jnp.int32, (1, _N), 1)

    for b in range(4):
        copies[2 * b].wait()
        copies[2 * b + 1].wait()
        pooled = jnp.concatenate(
            [jnp.max(buf[2 * b], axis=(1, 2)),
             jnp.max(buf[2 * b + 1], axis=(1, 2))])[None, :]

        h = jax.nn.relu(jnp.dot(pooled, w1t_ref[...],
                                preferred_element_type=jnp.float32)
                        + b1_ref[...])
        h = jax.nn.relu(jnp.dot(h, w2t_ref[...],
                                preferred_element_type=jnp.float32)
                        + b2_ref[...])
        p = jnp.dot(h, wpt_ref[...], preferred_element_type=jnp.float32)

        pred_ln = _ln(p) / 10.0
        logits = pred_ln + ce_ln
        cand = jnp.where(col == ic, 0.0, keep_ref[0, b, :][None, :])
        masked_exp = jnp.exp(logits) * cand
        y_soft = masked_exp / (jnp.sum(masked_exp, axis=-1, keepdims=True)
                               + 1e-8)

        max_v = jnp.max(y_soft, axis=-1, keepdims=True)
        idx_b = jnp.min(jnp.where(y_soft == max_v, col, _N))
        y_hard = (col == idx_b).astype(jnp.float32)
        cph = y_hard - y_soft + y_soft

        ce_ln_ref[0, b, :] = ce_ln[0]
        pred_ln_ref[0, b, :] = pred_ln[0]
        cph_ref[0, b, :] = cph[0]
        idx_ref[b] = idx_b


def _combine_body(ic_ref, idx_ref, wf_any, out_any, sems):
    ic = ic_ref[0]
    copies = []
    for job in range(8):
        b, sel = job % 4, job // 4
        cam = jnp.where(sel == 0, ic, idx_ref[b])
        cp = pltpu.make_async_copy(
            wf_any.at[b, cam], out_any.at[b, sel], sems.at[job])
        cp.start()
        copies.append(cp)
    for cp in copies:
        cp.wait()


def kernel(init_cam, world_feat, keep_cams, cam_emb, W1, b1, W2, b2, Wp):
    B, N, C, H, W = world_feat.shape
    ic_arr = jnp.asarray(init_cam, jnp.int32).reshape(1)
    keep_f = keep_cams.astype(jnp.float32).reshape(1, B, N)

    ce_ln3, pred_ln3, cph3, idx = pl.pallas_call(
        _route_body,
        grid_spec=pltpu.PrefetchScalarGridSpec(
            num_scalar_prefetch=1,
            grid=(1,),
            in_specs=[
                pl.BlockSpec(memory_space=pltpu.MemorySpace.HBM),
                pl.BlockSpec((1, B, N), lambda i, ic: (0, 0, 0)),
                pl.BlockSpec((N, N), lambda i, ic: (0, 0)),
                pl.BlockSpec((C, C), lambda i, ic: (0, 0)),
                pl.BlockSpec((1, C), lambda i, ic: (0, 0)),
                pl.BlockSpec((C, C), lambda i, ic: (0, 0)),
                pl.BlockSpec((1, C), lambda i, ic: (0, 0)),
                pl.BlockSpec((C, N), lambda i, ic: (0, 0)),
            ],
            out_specs=[
                pl.BlockSpec((1, B, N), lambda i, ic: (0, 0, 0)),
                pl.BlockSpec((1, B, N), lambda i, ic: (0, 0, 0)),
                pl.BlockSpec((1, B, N), lambda i, ic: (0, 0, 0)),
                pl.BlockSpec(memory_space=pltpu.SMEM),
            ],
            scratch_shapes=[
                pltpu.VMEM((8, _CH, H, W), jnp.float32),
                pltpu.SemaphoreType.DMA((8,)),
            ],
        ),
        out_shape=[
            jax.ShapeDtypeStruct((1, B, N), jnp.float32),
            jax.ShapeDtypeStruct((1, B, N), jnp.float32),
            jax.ShapeDtypeStruct((1, B, N), jnp.float32),
            jax.ShapeDtypeStruct((B,), jnp.int32),
        ],
    )(ic_arr, world_feat, keep_f, cam_emb, W1.T, b1.reshape(1, C), W2.T,
      b2.reshape(1, C), Wp.T)

    out = pl.pallas_call(
        _combine_body,
        grid_spec=pltpu.PrefetchScalarGridSpec(
            num_scalar_prefetch=2,
            grid=(1,),
            in_specs=[pl.BlockSpec(memory_space=pltpu.MemorySpace.HBM)],
            out_specs=pl.BlockSpec(memory_space=pltpu.MemorySpace.HBM),
            scratch_shapes=[pltpu.SemaphoreType.DMA((8,))],
        ),
        out_shape=jax.ShapeDtypeStruct((B, 2, C, H, W), jnp.float32),
    )(ic_arr, idx, world_feat)

    return (out, (ce_ln3.reshape(B, N), pred_ln3.reshape(B, N),
                  cph3.reshape(B, N)))


# speedup vs baseline: 1.0024x; 1.0024x over previous
import functools

import jax
import jax.numpy as jnp
from jax.experimental import pallas as pl
from jax.experimental.pallas import tpu as pltpu
from jax.experimental.pallas import tpu_sc as plsc

_N = 8
_C = 128
_CB = 64
_CQ = 32


def _route_body(ic_ref, wf0_ref, wf1_ref, wf2_ref, wf3_ref, keep_ref,
                cam_emb_ref, w1t_ref, b1_ref, w2t_ref, b2_ref, wpt_ref,
                ce_ln_ref, pred_ln_ref, cph_ref, idx_ref):
    b = pl.program_id(0)
    ic = ic_ref[0]

    quarters = [
        jnp.max(r[0, 0, :, :, :], axis=(1, 2))[None, :]
        for r in (wf0_ref, wf1_ref, wf2_ref, wf3_ref)
    ]
    pooled = jnp.concatenate(quarters, axis=1)

    h = jax.nn.relu(jnp.dot(pooled, w1t_ref[...],
                            preferred_element_type=jnp.float32) + b1_ref[...])
    h = jax.nn.relu(jnp.dot(h, w2t_ref[...],
                            preferred_element_type=jnp.float32) + b2_ref[...])
    p = jnp.dot(h, wpt_ref[...], preferred_element_type=jnp.float32)

    def _ln(v):
        m = jnp.mean(v, axis=-1, keepdims=True)
        var = jnp.mean((v - m) ** 2, axis=-1, keepdims=True)
        return (v - m) / jnp.sqrt(var + 1e-5)

    pred_ln = _ln(p) / 10.0

    ce = cam_emb_ref[...]
    row_sel = (jax.lax.broadcasted_iota(jnp.int32, (_N, 1), 0) == ic)
    ce_row = jnp.sum(jnp.where(row_sel, ce, 0.0), axis=0)[None, :]
    ce_ln = _ln(ce_row)

    logits = pred_ln + ce_ln
    col = jax.lax.broadcasted_iota(jnp.int32, (1, _N), 1)
    cand = jnp.where(col == ic, 0.0, keep_ref[0, 0, :][None, :])
    masked_exp = jnp.exp(logits) * cand
    y_soft = masked_exp / (jnp.sum(masked_exp, axis=-1, keepdims=True) + 1e-8)

    max_v = jnp.max(y_soft, axis=-1, keepdims=True)
    idx_b = jnp.min(jnp.where(y_soft == max_v, col, _N))
    y_hard = (col == idx_b).astype(jnp.float32)
    cph = y_hard - y_soft + y_soft

    ce_ln_ref[...] = ce_ln.reshape(1, 1, _N)
    pred_ln_ref[...] = pred_ln.reshape(1, 1, _N)
    cph_ref[...] = cph.reshape(1, 1, _N)
    idx_ref[b] = idx_b


def _sc_combine_body(wf_ref, rows_ref, out_ref, rows_s, sems):
    c = jax.lax.axis_index("c")
    pltpu.sync_copy(rows_ref, rows_s)
    copies = []
    for job in range(4):
        j = c * 4 + job
        src = rows_s[j]
        dst = jnp.where(j < 4, 2 * j, 2 * j - 7)
        cp = pltpu.make_async_copy(wf_ref.at[src], out_ref.at[dst],
                                   sems.at[job])
        cp.start()
        copies.append(cp)
    for cp in copies:
        cp.wait()


def kernel(init_cam, world_feat, keep_cams, cam_emb, W1, b1, W2, b2, Wp):
    B, N, C, H, W = world_feat.shape
    ic_arr = jnp.asarray(init_cam, jnp.int32).reshape(1)
    keep_f = keep_cams.astype(jnp.float32).reshape(B, 1, N)

    def _q_spec(q):
        return pl.BlockSpec((1, 1, _CQ, H, W),
                            lambda b, ic, q=q: (b, ic[0], q, 0, 0))

    ce_ln3, pred_ln3, cph3, idx = pl.pallas_call(
        _route_body,
        grid_spec=pltpu.PrefetchScalarGridSpec(
            num_scalar_prefetch=1,
            grid=(B,),
            in_specs=[
                _q_spec(0), _q_spec(1), _q_spec(2), _q_spec(3),
                pl.BlockSpec((1, 1, N), lambda b, ic: (b, 0, 0)),
                pl.BlockSpec((N, N), lambda b, ic: (0, 0)),
                pl.BlockSpec((C, C), lambda b, ic: (0, 0)),
                pl.BlockSpec((1, C), lambda b, ic: (0, 0)),
                pl.BlockSpec((C, C), lambda b, ic: (0, 0)),
                pl.BlockSpec((1, C), lambda b, ic: (0, 0)),
                pl.BlockSpec((C, N), lambda b, ic: (0, 0)),
            ],
            out_specs=[
                pl.BlockSpec((1, 1, N), lambda b, ic: (b, 0, 0)),
                pl.BlockSpec((1, 1, N), lambda b, ic: (b, 0, 0)),
                pl.BlockSpec((1, 1, N), lambda b, ic: (b, 0, 0)),
                pl.BlockSpec(memory_space=pltpu.SMEM),
            ],
        ),
        out_shape=[
            jax.ShapeDtypeStruct((B, 1, N), jnp.float32),
            jax.ShapeDtypeStruct((B, 1, N), jnp.float32),
            jax.ShapeDtypeStruct((B, 1, N), jnp.float32),
            jax.ShapeDtypeStruct((B,), jnp.int32),
        ],
    )(ic_arr, world_feat, world_feat, world_feat, world_feat, keep_f,
      cam_emb, W1.T, b1.reshape(1, C), W2.T, b2.reshape(1, C), Wp.T)

    bidx = jnp.arange(B, dtype=jnp.int32)
    ic_s = jnp.asarray(init_cam, jnp.int32)
    rows = jnp.concatenate(
        [bidx * N + ic_s, bidx * N + idx,
         jnp.zeros((8,), jnp.int32)]).astype(jnp.int32)

    wf2 = world_feat.reshape(B * N, C, H, W)
    out2 = functools.partial(
        pl.kernel,
        out_type=jax.ShapeDtypeStruct((B * 2, C, H, W), jnp.float32),
        mesh=plsc.ScalarSubcoreMesh(axis_name="c", num_cores=2),
        scratch_types=[pltpu.SMEM((16,), jnp.int32),
                       pltpu.SemaphoreType.DMA((4,))],
    )(_sc_combine_body)(wf2, rows)
    out = out2.reshape(B, 2, C, H, W)

    return (out, (ce_ln3.reshape(B, N), pred_ln3.reshape(B, N),
                  cph3.reshape(B, N)))


# speedup vs baseline: 6.6314x; 6.6155x over previous
import jax
import jax.numpy as jnp
from jax.experimental import pallas as pl
from jax.experimental.pallas import tpu as pltpu

_N = 8
_C = 128
_CB = 64
_CQ = 32


def _route_body(ic_ref, wf0_ref, wf1_ref, wf2_ref, wf3_ref, keep_ref,
                cam_emb_ref, w1t_ref, b1_ref, w2t_ref, b2_ref, wpt_ref,
                ce_ln_ref, pred_ln_ref, cph_ref, idx_ref, out_ref):
    b = pl.program_id(0)
    ic = ic_ref[0]

    refs = (wf0_ref, wf1_ref, wf2_ref, wf3_ref)
    quarters = [
        jnp.max(r[0, 0, :, :, :], axis=(1, 2))[None, :]
        for r in refs
    ]
    pooled = jnp.concatenate(quarters, axis=1)
    for q, r in enumerate(refs):
        out_ref[0, 0, q * _CQ:(q + 1) * _CQ, :, :] = r[0, 0, :, :, :]

    h = jax.nn.relu(jnp.dot(pooled, w1t_ref[...],
                            preferred_element_type=jnp.float32) + b1_ref[...])
    h = jax.nn.relu(jnp.dot(h, w2t_ref[...],
                            preferred_element_type=jnp.float32) + b2_ref[...])
    p = jnp.dot(h, wpt_ref[...], preferred_element_type=jnp.float32)

    def _ln(v):
        m = jnp.mean(v, axis=-1, keepdims=True)
        var = jnp.mean((v - m) ** 2, axis=-1, keepdims=True)
        return (v - m) / jnp.sqrt(var + 1e-5)

    pred_ln = _ln(p) / 10.0

    ce = cam_emb_ref[...]
    row_sel = (jax.lax.broadcasted_iota(jnp.int32, (_N, 1), 0) == ic)
    ce_row = jnp.sum(jnp.where(row_sel, ce, 0.0), axis=0)[None, :]
    ce_ln = _ln(ce_row)

    logits = pred_ln + ce_ln
    col = jax.lax.broadcasted_iota(jnp.int32, (1, _N), 1)
    cand = jnp.where(col == ic, 0.0, keep_ref[0, 0, :][None, :])
    masked_exp = jnp.exp(logits) * cand
    y_soft = masked_exp / (jnp.sum(masked_exp, axis=-1, keepdims=True) + 1e-8)

    max_v = jnp.max(y_soft, axis=-1, keepdims=True)
    idx_b = jnp.min(jnp.where(y_soft == max_v, col, _N))
    y_hard = (col == idx_b).astype(jnp.float32)
    cph = y_hard - y_soft + y_soft

    ce_ln_ref[...] = ce_ln.reshape(1, 1, _N)
    pred_ln_ref[...] = pred_ln.reshape(1, 1, _N)
    cph_ref[...] = cph.reshape(1, 1, _N)
    idx_ref[b] = idx_b


def _copy_body(ic_ref, idx_ref, wf_ref, cph_ref, prev_ref, out_ref):
    out_ref[...] = wf_ref[...] * jnp.sum(cph_ref[...])


def kernel(init_cam, world_feat, keep_cams, cam_emb, W1, b1, W2, b2, Wp):
    B, N, C, H, W = world_feat.shape
    ic_arr = jnp.asarray(init_cam, jnp.int32).reshape(1)
    keep_f = keep_cams.astype(jnp.float32).reshape(B, 1, N)

    def _q_spec(q):
        return pl.BlockSpec((1, 1, _CQ, H, W),
                            lambda b, ic, q=q: (b, ic[0], q, 0, 0))

    ce_ln3, pred_ln3, cph3, idx, out_half = pl.pallas_call(
        _route_body,
        grid_spec=pltpu.PrefetchScalarGridSpec(
            num_scalar_prefetch=1,
            grid=(B,),
            in_specs=[
                _q_spec(0), _q_spec(1), _q_spec(2), _q_spec(3),
                pl.BlockSpec((1, 1, N), lambda b, ic: (b, 0, 0)),
                pl.BlockSpec((N, N), lambda b, ic: (0, 0)),
                pl.BlockSpec((C, C), lambda b, ic: (0, 0)),
                pl.BlockSpec((1, C), lambda b, ic: (0, 0)),
                pl.BlockSpec((C, C), lambda b, ic: (0, 0)),
                pl.BlockSpec((1, C), lambda b, ic: (0, 0)),
                pl.BlockSpec((C, N), lambda b, ic: (0, 0)),
            ],
            out_specs=[
                pl.BlockSpec((1, 1, N), lambda b, ic: (b, 0, 0)),
                pl.BlockSpec((1, 1, N), lambda b, ic: (b, 0, 0)),
                pl.BlockSpec((1, 1, N), lambda b, ic: (b, 0, 0)),
                pl.BlockSpec(memory_space=pltpu.SMEM),
                pl.BlockSpec((1, 1, C, H, W),
                             lambda b, ic: (b, 0, 0, 0, 0)),
            ],
        ),
        out_shape=[
            jax.ShapeDtypeStruct((B, 1, N), jnp.float32),
            jax.ShapeDtypeStruct((B, 1, N), jnp.float32),
            jax.ShapeDtypeStruct((B, 1, N), jnp.float32),
            jax.ShapeDtypeStruct((B,), jnp.int32),
            jax.ShapeDtypeStruct((B, 2, C, H, W), jnp.float32),
        ],
    )(ic_arr, world_feat, world_feat, world_feat, world_feat, keep_f,
      cam_emb, W1.T, b1.reshape(1, C), W2.T, b2.reshape(1, C), Wp.T)

    out = pl.pallas_call(
        _copy_body,
        grid_spec=pltpu.PrefetchScalarGridSpec(
            num_scalar_prefetch=2,
            grid=(B, C // _CB),
            in_specs=[
                pl.BlockSpec(
                    (1, 1, _CB, H, W),
                    lambda b, c, ic, idx: (b, idx[b], c, 0, 0),
                ),
                pl.BlockSpec((1, 1, N), lambda b, c, ic, idx: (b, 0, 0)),
                pl.BlockSpec(memory_space=pltpu.MemorySpace.HBM),
            ],
            out_specs=pl.BlockSpec(
                (1, 1, _CB, H, W),
                lambda b, c, ic, idx: (b, 1, c, 0, 0),
            ),
        ),
        out_shape=jax.ShapeDtypeStruct((B, 2, C, H, W), jnp.float32),
        input_output_aliases={4: 0},
    )(ic_arr, idx, world_feat, cph3, out_half)

    return (out, (ce_ln3.reshape(B, N), pred_ln3.reshape(B, N),
                  cph3.reshape(B, N)))
